# baseline (speedup 1.0000x reference)
import functools

import jax
import jax.numpy as jnp
from jax import lax
from jax.experimental import pallas as pl
from jax.experimental.pallas import tpu as pltpu
from jax.experimental.pallas import tpu_sc as plsc

D = 64
NW = 32
VB = 16384
LG = 4
BB = 512
GGRP = 4


def _pack_body(t_ref, o_ref):
    u = lax.bitcast_convert_type(t_ref[...], jnp.uint32)
    r = (u + jnp.uint32(0x7FFF) + ((u >> 16) & jnp.uint32(1))) >> 16
    w = r[:32, :] | (r[32:, :] << 16)
    vb2 = VB // 2
    parts = [jnp.transpose(w[:, k * vb2:(k + 1) * vb2]) for k in range(2)]
    o_ref[...] = lax.bitcast_convert_type(
        jnp.concatenate(parts, axis=1), jnp.float32)


def _sc_gather_body(n_l, table_ref, idx_ref, out_ref, idx_v, rows_a, rows_b, sem):
    wid = lax.axis_index("s") * 2 + lax.axis_index("c")
    b0 = wid * 128
    pltpu.sync_copy(idx_ref.at[:, pl.ds(b0, 128)], idx_v)
    n_grp = n_l // GGRP

    def fire(grp, buf):
        for j in range(GGRP):
            pltpu.async_copy(
                table_ref.at[idx_v.at[grp * GGRP + j]],
                buf.at[pl.ds(j * 128, 128), :],
                sem,
            )

    def drain_and_write(grp, buf):
        pltpu.make_async_copy(
            out_ref.at[pl.ds(0, GGRP * 128), pl.ds(0, 64)], buf, sem
        ).wait()
        for j in range(GGRP):
            pltpu.sync_copy(
                buf.at[pl.ds(j * 128, 128), :],
                out_ref.at[pl.ds(b0, 128),
                           pl.ds((grp * GGRP + j) * 64, 64)],
            )

    fire(0, rows_a)

    def body(h, carry):
        g = h * 2
        fire(g + 1, rows_b)
        drain_and_write(g, rows_a)

        @pl.when(g + 2 < n_grp)
        def _():
            fire(g + 2, rows_a)

        drain_and_write(g + 1, rows_b)
        return carry

    lax.fori_loop(0, n_grp // 2, body, 0)


def _fuse_body(g_ref, n_ref, m_ref, wf_ref, wn_ref, bn_ref, bf_ref, o_ref):
    wf = wf_ref[...]
    w1 = wf[:, :D]
    w2 = wf[:, D:]
    cd = (((1,), (0,)), ((), ()))
    v = lax.dot_general(w2, wn_ref[...], cd,
                        preferred_element_type=jnp.float32)
    c = lax.dot_general(w2, bn_ref[...], cd,
                        preferred_element_type=jnp.float32) + bf_ref[...]
    zero = jnp.zeros((D, D), jnp.float32)
    w4 = jnp.concatenate(
        [jnp.concatenate([w1 if q == p else zero for q in range(LG)], axis=1)
         for p in range(LG)], axis=0)

    gu = lax.bitcast_convert_type(g_ref[...], jnp.uint32)
    mq = m_ref[...].reshape(LG, BB)
    halves = []
    for p in range(LG):
        gp = gu[:, p * 64:(p + 1) * 64]
        m = jnp.transpose(mq[p:p + 1, :])
        sl = jnp.where(m == 0, gp[:, :32], gp[:, 32:])
        halves.append(lax.bitcast_convert_type(sl << 16, jnp.float32))
        halves.append(lax.bitcast_convert_type(
            sl & jnp.uint32(0xFFFF0000), jnp.float32))
    gf = jnp.concatenate(halves, axis=1)
    out = lax.dot_general(w4, gf, (((1,), (1,)), ((), ())),
                          preferred_element_type=jnp.float32)

    n4 = n_ref[...].reshape(LG, BB)
    nx = jnp.broadcast_to(n4[:, None, :], (LG, D, BB)).reshape(LG * D, BB)
    v4 = jnp.concatenate([v] * LG, axis=0)
    c4 = jnp.concatenate([c] * LG, axis=0)
    o_ref[...] = (out + v4 * nx + c4).reshape(LG, D, BB)


def kernel(categorical_features, numeric_features, emb_table, W_num, b_num, W_fuse, b_fuse):
    b, n_l = categorical_features.shape
    vocab = emb_table.shape[0]

    t_phys = emb_table.T
    idx_t = categorical_features.T
    num_t = numeric_features.T

    n_blk = pl.cdiv(vocab, VB)
    packed = pl.pallas_call(
        _pack_body,
        grid=(n_blk,),
        in_specs=[pl.BlockSpec((D, VB), lambda i: (0, i))],
        out_specs=pl.BlockSpec((VB // 2, D), lambda i: (i, 0)),
        out_shape=jax.ShapeDtypeStruct((n_blk * (VB // 2), D), jnp.float32),
    )(t_phys)

    vb2 = VB // 2
    idx2 = ((idx_t // VB) * vb2) | (idx_t & (vb2 - 1))
    idxm = ((idx_t // vb2) & 1).reshape(n_l // LG, LG, b)

    mesh = plsc.VectorSubcoreMesh(core_axis_name="c", subcore_axis_name="s")
    gathered = pl.kernel(
        functools.partial(_sc_gather_body, n_l),
        out_type=jax.ShapeDtypeStruct((b, n_l * D), jnp.float32),
        mesh=mesh,
        scratch_types=[
            pltpu.VMEM((n_l, 128), jnp.int32),
            pltpu.VMEM((GGRP * 128, D), jnp.float32),
            pltpu.VMEM((GGRP * 128, D), jnp.float32),
            pltpu.SemaphoreType.DMA,
        ],
        compiler_params=pltpu.CompilerParams(use_tc_tiling_on_sc=False),
    )(packed, idx2)

    num_r = num_t.reshape(n_l // LG, LG, b)
    out_t = pl.pallas_call(
        _fuse_body,
        grid=(b // BB, n_l // LG),
        in_specs=[
            pl.BlockSpec((BB, LG * D), lambda ib, il: (ib, il)),
            pl.BlockSpec((1, LG, BB), lambda ib, il: (il, 0, ib)),
            pl.BlockSpec((1, LG, BB), lambda ib, il: (il, 0, ib)),
            pl.BlockSpec((D, 2 * D), lambda ib, il: (0, 0)),
            pl.BlockSpec((D, 1), lambda ib, il: (0, 0)),
            pl.BlockSpec((D, 1), lambda ib, il: (0, 0)),
            pl.BlockSpec((D, 1), lambda ib, il: (0, 0)),
        ],
        out_specs=pl.BlockSpec((LG, D, BB), lambda ib, il: (il, 0, ib)),
        out_shape=jax.ShapeDtypeStruct((n_l, D, b), jnp.float32),
    )(gathered, num_r, idxm, W_fuse, W_num, b_num.reshape(D, 1),
      b_fuse.reshape(D, 1))

    return jnp.transpose(out_t, (2, 0, 1))


# speedup vs baseline: 1.0731x; 1.0731x over previous
import functools

import jax
import jax.numpy as jnp
from jax import lax
from jax.experimental import pallas as pl
from jax.experimental.pallas import tpu as pltpu
from jax.experimental.pallas import tpu_sc as plsc

D = 64
NW = 32
VB = 16384
LG = 4
BB = 1024
GGRP = 4


def _pack_body(t_ref, o_ref):
    u = lax.bitcast_convert_type(t_ref[...], jnp.uint32)
    r = (u + jnp.uint32(0x7FFF) + ((u >> 16) & jnp.uint32(1))) >> 16
    w = r[:32, :] | (r[32:, :] << 16)
    vb2 = VB // 2
    parts = [jnp.transpose(w[:, k * vb2:(k + 1) * vb2]) for k in range(2)]
    o_ref[...] = lax.bitcast_convert_type(
        jnp.concatenate(parts, axis=1), jnp.float32)


def _sc_gather_body(n_l, table_ref, idx_ref, out_ref, idx_v, rows_a, rows_b, sem):
    wid = lax.axis_index("s") * 2 + lax.axis_index("c")
    b0 = wid * 128
    pltpu.sync_copy(idx_ref.at[:, pl.ds(b0, 128)], idx_v)
    n_grp = n_l // GGRP

    def fire(grp, buf):
        for j in range(GGRP):
            pltpu.async_copy(
                table_ref.at[idx_v.at[grp * GGRP + j]],
                buf.at[pl.ds(j * 128, 128), :],
                sem,
            )

    def drain_and_write(grp, buf):
        pltpu.make_async_copy(
            out_ref.at[pl.ds(0, GGRP * 128), pl.ds(0, 64)], buf, sem
        ).wait()
        for j in range(GGRP):
            pltpu.sync_copy(
                buf.at[pl.ds(j * 128, 128), :],
                out_ref.at[pl.ds(b0, 128),
                           pl.ds((grp * GGRP + j) * 64, 64)],
            )

    fire(0, rows_a)

    def body(h, carry):
        g = h * 2
        fire(g + 1, rows_b)
        drain_and_write(g, rows_a)

        @pl.when(g + 2 < n_grp)
        def _():
            fire(g + 2, rows_a)

        drain_and_write(g + 1, rows_b)
        return carry

    lax.fori_loop(0, n_grp // 2, body, 0)


def _fuse_body(g_ref, n_ref, m_ref, wf_ref, wn_ref, bn_ref, bf_ref, o_ref):
    wf = wf_ref[...]
    w1 = wf[:, :D]
    w2 = wf[:, D:]
    cd = (((1,), (0,)), ((), ()))
    v = lax.dot_general(w2, wn_ref[...], cd,
                        preferred_element_type=jnp.float32)
    c = lax.dot_general(w2, bn_ref[...], cd,
                        preferred_element_type=jnp.float32) + bf_ref[...]
    zero = jnp.zeros((D, D), jnp.float32)
    w4 = jnp.concatenate(
        [jnp.concatenate([w1 if q == p else zero for q in range(LG)], axis=1)
         for p in range(LG)], axis=0)

    gu = lax.bitcast_convert_type(g_ref[...], jnp.uint32)
    mq = m_ref[...].reshape(LG, BB)
    halves = []
    for p in range(LG):
        gp = gu[:, p * 64:(p + 1) * 64]
        m = jnp.transpose(mq[p:p + 1, :])
        sl = jnp.where(m == 0, gp[:, :32], gp[:, 32:])
        halves.append(lax.bitcast_convert_type(sl << 16, jnp.float32))
        halves.append(lax.bitcast_convert_type(
            sl & jnp.uint32(0xFFFF0000), jnp.float32))
    gf = jnp.concatenate(halves, axis=1)
    out = lax.dot_general(w4, gf, (((1,), (1,)), ((), ())),
                          preferred_element_type=jnp.float32)

    n4 = n_ref[...].reshape(LG, BB)
    nx = jnp.broadcast_to(n4[:, None, :], (LG, D, BB)).reshape(LG * D, BB)
    v4 = jnp.concatenate([v] * LG, axis=0)
    c4 = jnp.concatenate([c] * LG, axis=0)
    o_ref[...] = (out + v4 * nx + c4).reshape(LG, D, BB)


def kernel(categorical_features, numeric_features, emb_table, W_num, b_num, W_fuse, b_fuse):
    b, n_l = categorical_features.shape
    vocab = emb_table.shape[0]

    t_phys = emb_table.T
    idx_t = categorical_features.T
    num_t = numeric_features.T

    n_blk = pl.cdiv(vocab, VB)
    packed = pl.pallas_call(
        _pack_body,
        grid=(n_blk,),
        in_specs=[pl.BlockSpec((D, VB), lambda i: (0, i))],
        out_specs=pl.BlockSpec((VB // 2, D), lambda i: (i, 0)),
        out_shape=jax.ShapeDtypeStruct((n_blk * (VB // 2), D), jnp.float32),
    )(t_phys)

    vb2 = VB // 2
    idx2 = ((idx_t // VB) * vb2) | (idx_t & (vb2 - 1))
    idxm = ((idx_t // vb2) & 1).reshape(n_l // LG, LG, b)

    mesh = plsc.VectorSubcoreMesh(core_axis_name="c", subcore_axis_name="s")
    gathered = pl.kernel(
        functools.partial(_sc_gather_body, n_l),
        out_type=jax.ShapeDtypeStruct((b, n_l * D), jnp.float32),
        mesh=mesh,
        scratch_types=[
            pltpu.VMEM((n_l, 128), jnp.int32),
            pltpu.VMEM((GGRP * 128, D), jnp.float32),
            pltpu.VMEM((GGRP * 128, D), jnp.float32),
            pltpu.SemaphoreType.DMA,
        ],
        compiler_params=pltpu.CompilerParams(use_tc_tiling_on_sc=False),
    )(packed, idx2)

    num_r = num_t.reshape(n_l // LG, LG, b)
    out_t = pl.pallas_call(
        _fuse_body,
        grid=(b // BB, n_l // LG),
        in_specs=[
            pl.BlockSpec((BB, LG * D), lambda ib, il: (ib, il)),
            pl.BlockSpec((1, LG, BB), lambda ib, il: (il, 0, ib)),
            pl.BlockSpec((1, LG, BB), lambda ib, il: (il, 0, ib)),
            pl.BlockSpec((D, 2 * D), lambda ib, il: (0, 0)),
            pl.BlockSpec((D, 1), lambda ib, il: (0, 0)),
            pl.BlockSpec((D, 1), lambda ib, il: (0, 0)),
            pl.BlockSpec((D, 1), lambda ib, il: (0, 0)),
        ],
        out_specs=pl.BlockSpec((LG, D, BB), lambda ib, il: (il, 0, ib)),
        out_shape=jax.ShapeDtypeStruct((n_l, D, b), jnp.float32),
    )(gathered, num_r, idxm, W_fuse, W_num, b_num.reshape(D, 1),
      b_fuse.reshape(D, 1))

    return jnp.transpose(out_t, (2, 0, 1))


# speedup vs baseline: 1.1743x; 1.0943x over previous
import functools

import jax
import jax.numpy as jnp
from jax import lax
from jax.experimental import pallas as pl
from jax.experimental.pallas import tpu as pltpu
from jax.experimental.pallas import tpu_sc as plsc

D = 64
NW = 32
VB = 16384
LG = 4
BB = 1024
GGRP = 2


def _pack_body(t_ref, o_ref):
    u = lax.bitcast_convert_type(t_ref[...], jnp.uint32)
    r = (u + jnp.uint32(0x7FFF) + ((u >> 16) & jnp.uint32(1))) >> 16
    w = r[:32, :] | (r[32:, :] << 16)
    vb4 = VB // 4
    parts = [jnp.transpose(w[:, j * vb4:(j + 1) * vb4]) for j in range(4)]
    o_ref[...] = lax.bitcast_convert_type(
        jnp.concatenate(parts, axis=1), jnp.float32)


def _sc_gather_body(n_l, table_ref, idx_ref, out_ref, idx_v, rows_a, rows_b, sem):
    wid = lax.axis_index("s") * 2 + lax.axis_index("c")
    b0 = wid * 128
    pltpu.sync_copy(idx_ref.at[:, pl.ds(b0, 128)], idx_v)
    n_grp = n_l // GGRP
    gcols = GGRP * 128

    def fire(grp, buf):
        for j in range(GGRP):
            pltpu.async_copy(
                table_ref.at[idx_v.at[grp * GGRP + j]],
                buf.at[:, pl.ds(j * 128, 128)],
                sem,
            )

    def drain_and_write(grp, buf):
        pltpu.make_async_copy(
            out_ref.at[pl.ds(0, 128), pl.ds(0, gcols)], buf, sem
        ).wait()
        pltpu.sync_copy(
            buf, out_ref.at[pl.ds(b0, 128), pl.ds(grp * gcols, gcols)]
        )

    fire(0, rows_a)

    def body(h, carry):
        g = h * 2
        fire(g + 1, rows_b)
        drain_and_write(g, rows_a)

        @pl.when(g + 2 < n_grp)
        def _():
            fire(g + 2, rows_a)

        drain_and_write(g + 1, rows_b)
        return carry

    lax.fori_loop(0, n_grp // 2, body, 0)


def _fuse_body(g_ref, n_ref, m_ref, wf_ref, wn_ref, bn_ref, bf_ref, o_ref):
    wf = wf_ref[...]
    w1 = wf[:, :D]
    w2 = wf[:, D:]
    cd = (((1,), (0,)), ((), ()))
    v = lax.dot_general(w2, wn_ref[...], cd,
                        preferred_element_type=jnp.float32)
    c = lax.dot_general(w2, bn_ref[...], cd,
                        preferred_element_type=jnp.float32) + bf_ref[...]
    zero = jnp.zeros((D, D), jnp.float32)
    w4 = jnp.concatenate(
        [jnp.concatenate([w1 if q == p else zero for q in range(LG)], axis=1)
         for p in range(LG)], axis=0)

    gu = lax.bitcast_convert_type(g_ref[...], jnp.uint32)
    mq = m_ref[...].reshape(LG, BB)
    halves = []
    for p in range(LG):
        gp = gu[:, p * 128:(p + 1) * 128]
        m = mq[p][:, None]
        sl = jnp.where(
            m < 2,
            jnp.where(m == 0, gp[:, 0:32], gp[:, 32:64]),
            jnp.where(m == 2, gp[:, 64:96], gp[:, 96:128]),
        )
        halves.append(lax.bitcast_convert_type(sl << 16, jnp.float32))
        halves.append(lax.bitcast_convert_type(
            sl & jnp.uint32(0xFFFF0000), jnp.float32))
    gf = jnp.concatenate(halves, axis=1)
    out = lax.dot_general(w4, gf, (((1,), (1,)), ((), ())),
                          preferred_element_type=jnp.float32)

    n4 = n_ref[...].reshape(LG, BB)
    nx = jnp.broadcast_to(n4[:, None, :], (LG, D, BB)).reshape(LG * D, BB)
    v4 = jnp.concatenate([v] * LG, axis=0)
    c4 = jnp.concatenate([c] * LG, axis=0)
    o_ref[...] = (out + v4 * nx + c4).reshape(LG, D, BB)


def kernel(categorical_features, numeric_features, emb_table, W_num, b_num, W_fuse, b_fuse):
    b, n_l = categorical_features.shape
    vocab = emb_table.shape[0]

    t_phys = emb_table.T
    idx_t = categorical_features.T
    num_t = numeric_features.T

    n_blk = pl.cdiv(vocab, VB)
    packed = pl.pallas_call(
        _pack_body,
        grid=(n_blk,),
        in_specs=[pl.BlockSpec((D, VB), lambda i: (0, i))],
        out_specs=pl.BlockSpec((VB // 4, 128), lambda i: (i, 0)),
        out_shape=jax.ShapeDtypeStruct((n_blk * (VB // 4), 128), jnp.float32),
    )(t_phys)

    vb4 = VB // 4
    idx4 = ((idx_t // VB) * vb4) | (idx_t & (vb4 - 1))
    idxm = ((idx_t // vb4) & 3).reshape(n_l // LG, LG, b)

    mesh = plsc.VectorSubcoreMesh(core_axis_name="c", subcore_axis_name="s")
    gathered = pl.kernel(
        functools.partial(_sc_gather_body, n_l),
        out_type=jax.ShapeDtypeStruct((b, n_l * 128), jnp.float32),
        mesh=mesh,
        scratch_types=[
            pltpu.VMEM((n_l, 128), jnp.int32),
            pltpu.VMEM((128, GGRP * 128), jnp.float32),
            pltpu.VMEM((128, GGRP * 128), jnp.float32),
            pltpu.SemaphoreType.DMA,
        ],
        compiler_params=pltpu.CompilerParams(use_tc_tiling_on_sc=True),
    )(packed, idx4)

    num_r = num_t.reshape(n_l // LG, LG, b)
    out_t = pl.pallas_call(
        _fuse_body,
        grid=(b // BB, n_l // LG),
        in_specs=[
            pl.BlockSpec((BB, LG * 128), lambda ib, il: (ib, il)),
            pl.BlockSpec((1, LG, BB), lambda ib, il: (il, 0, ib)),
            pl.BlockSpec((1, LG, BB), lambda ib, il: (il, 0, ib)),
            pl.BlockSpec((D, 2 * D), lambda ib, il: (0, 0)),
            pl.BlockSpec((D, 1), lambda ib, il: (0, 0)),
            pl.BlockSpec((D, 1), lambda ib, il: (0, 0)),
            pl.BlockSpec((D, 1), lambda ib, il: (0, 0)),
        ],
        out_specs=pl.BlockSpec((LG, D, BB), lambda ib, il: (il, 0, ib)),
        out_shape=jax.ShapeDtypeStruct((n_l, D, b), jnp.float32),
    )(gathered, num_r, idxm, W_fuse, W_num, b_num.reshape(D, 1),
      b_fuse.reshape(D, 1))

    return jnp.transpose(out_t, (2, 0, 1))
